# baseline (speedup 1.0000x reference)
import jax
import jax.numpy as jnp
from jax import lax
from jax.experimental import pallas as pl
from jax.experimental.pallas import tpu as pltpu

_NH = 32
_NKEEP = 1024
_N = 2048
_E = 256


def _gm_kernel(x_ref, gm_ref):
    gm_ref[...] = jnp.abs(jnp.mean(x_ref[...], axis=0, keepdims=True))


def _gm(x):
    return pl.pallas_call(
        _gm_kernel,
        grid=(2,),
        in_specs=[pl.BlockSpec((None, _N, _E), lambda b: (b, 0, 0))],
        out_specs=pl.BlockSpec((None, 1, _E), lambda b: (b, 0, 0)),
        out_shape=jax.ShapeDtypeStruct((2, 1, _E), jnp.float32),
    )(x)


def _ratio_kernel(cT_ref, cC_ref, x_ref, gm_ref, ratio_ref, xnh_ref):
    ckx = cC_ref[0:1, :]
    cky = cC_ref[1:2, :]
    iota = lax.broadcasted_iota(jnp.int32, (1, _N), 1)
    gmr = gm_ref[...].reshape(1, _E)

    def qbody(q, carry):
        cq = cT_ref[pl.ds(q, 1), :]
        dx = cq[0:1, 0:1] - ckx
        dy = cq[0:1, 1:2] - cky
        d2 = dx * dx + dy * dy

        def kbody(k, d2c):
            m = jnp.min(d2c, axis=1, keepdims=True)
            j = jnp.min(jnp.where(d2c == m, iota, _N))
            xnh_ref[pl.ds(k, 1), :] = x_ref[pl.ds(j, 1), :]
            return jnp.where(iota == j, jnp.inf, d2c)

        lax.fori_loop(0, _NH, kbody, d2)
        v = xnh_ref[...]
        am = jnp.mean(v, axis=0, keepdims=True)
        cent = lax.square(lax.sub(v, am))
        ss = jnp.sum(cent, axis=0, keepdims=True)
        ls = jnp.sqrt(lax.div(ss, jnp.float32(_NH - 1)))
        ratio_ref[pl.ds(q, 1), :] = ls / gmr
        return carry

    lax.fori_loop(0, _N, qbody, 0)


def _ratio(cT, cC, x, gm):
    return pl.pallas_call(
        _ratio_kernel,
        grid=(2,),
        in_specs=[
            pl.BlockSpec((None, _N, 2), lambda b: (b, 0, 0)),
            pl.BlockSpec((None, 2, _N), lambda b: (b, 0, 0)),
            pl.BlockSpec((None, _N, _E), lambda b: (b, 0, 0)),
            pl.BlockSpec((None, 1, _E), lambda b: (b, 0, 0)),
        ],
        out_specs=pl.BlockSpec((None, _N, _E), lambda b: (b, 0, 0)),
        out_shape=jax.ShapeDtypeStruct((2, _N, _E), jnp.float32),
        scratch_shapes=[pltpu.VMEM((_NH, _E), jnp.float32)],
    )(cT, cC, x, gm)


def _select_kernel(ldR_ref, ldC_ref, x_ref, cT_ref, xo_ref, co_ref):
    ldrow = ldR_ref[0:1, :]
    iota = lax.broadcasted_iota(jnp.int32, (1, _N), 1)

    def body(i, carry):
        ldi = jnp.sum(ldC_ref[pl.ds(i, 1), :])
        beats = (ldrow > ldi) | ((ldrow == ldi) & (iota < i))
        r = jnp.sum(beats.astype(jnp.int32))

        @pl.when(r < _NKEEP)
        def _():
            xo_ref[pl.ds(r, 1), :] = x_ref[pl.ds(i, 1), :]
            co_ref[pl.ds(r, 1), :] = cT_ref[pl.ds(i, 1), :]

        return carry

    lax.fori_loop(0, _N, body, 0)


def _select(ld, x, cT):
    ldR = ld.reshape(2, 1, _N)
    ldC = ld.reshape(2, _N, 1)
    xo, co = pl.pallas_call(
        _select_kernel,
        grid=(2,),
        in_specs=[
            pl.BlockSpec((None, 1, _N), lambda b: (b, 0, 0)),
            pl.BlockSpec((None, _N, 1), lambda b: (b, 0, 0)),
            pl.BlockSpec((None, _N, _E), lambda b: (b, 0, 0)),
            pl.BlockSpec((None, _N, 2), lambda b: (b, 0, 0)),
        ],
        out_specs=[
            pl.BlockSpec((None, _NKEEP, _E), lambda b: (b, 0, 0)),
            pl.BlockSpec((None, _NKEEP, 2), lambda b: (b, 0, 0)),
        ],
        out_shape=[
            jax.ShapeDtypeStruct((2, _NKEEP, _E), jnp.float32),
            jax.ShapeDtypeStruct((2, _NKEEP, 2), jnp.float32),
        ],
    )(ldR, ldC, x, cT)
    return xo, jnp.transpose(co, (0, 2, 1))[:, :, :, None]


def kernel(x, coords):
    cT = jnp.transpose(coords[:, :, :, 0], (0, 2, 1))
    cC = coords[:, :, :, 0]
    gm = _gm(x)
    ratio = _ratio(cT, cC, x, gm)
    ld = jnp.sum(ratio, axis=-1)
    pad = coords[:, 0, :, 0] > 999.0
    ld = jnp.where(pad, ld - 10000.0, ld)
    x_out, coords_out = _select(ld, x, cT)
    return (x_out, coords_out, ld)


# speedup vs baseline: 6.9984x; 6.9984x over previous
import jax
import jax.numpy as jnp
from jax import lax
from jax.experimental import pallas as pl
from jax.experimental.pallas import tpu as pltpu

_NH = 32
_NKEEP = 1024
_N = 2048
_E = 256


def _gm_kernel(x_ref, gm_ref):
    gm_ref[...] = jnp.abs(jnp.mean(x_ref[...], axis=0, keepdims=True))


def _gm(x):
    return pl.pallas_call(
        _gm_kernel,
        grid=(2,),
        in_specs=[pl.BlockSpec((None, _N, _E), lambda b: (b, 0, 0))],
        out_specs=pl.BlockSpec((None, 1, _E), lambda b: (b, 0, 0)),
        out_shape=jax.ShapeDtypeStruct((2, 1, _E), jnp.float32),
    )(x)


_QG = 8


def _ratio_kernel(cT_ref, cC_ref, x_ref, gm_ref, ratio_ref, xnh_ref):
    ckx = cC_ref[0:1, :]
    cky = cC_ref[1:2, :]
    iota = lax.broadcasted_iota(jnp.int32, (_QG, _N), 1)
    gmr = gm_ref[...].reshape(1, _E)

    def gbody(g, carry):
        qbase = g * _QG
        cq = cT_ref[pl.ds(qbase, _QG), :]
        dx = cq[:, 0:1] - ckx
        dy = cq[:, 1:2] - cky
        d2 = dx * dx + dy * dy

        def kbody(k, d2c):
            m = jnp.min(d2c, axis=1, keepdims=True)
            cand = jnp.where(d2c == m, iota, _N)
            jcol = jnp.min(cand, axis=1, keepdims=True)
            for qq in range(_QG):
                j = jnp.min(jcol[qq:qq + 1, 0:1])
                xnh_ref[pl.ds(qq * _NH + k, 1), :] = x_ref[pl.ds(j, 1), :]
            return jnp.where(iota == jcol, jnp.inf, d2c)

        lax.fori_loop(0, _NH, kbody, d2)
        rows = []
        for qq in range(_QG):
            v = xnh_ref[qq * _NH:(qq + 1) * _NH, :]
            am = jnp.mean(v, axis=0, keepdims=True)
            cent = lax.square(lax.sub(v, am))
            ss = jnp.sum(cent, axis=0, keepdims=True)
            ls = jnp.sqrt(lax.div(ss, jnp.float32(_NH - 1)))
            rows.append(ls / gmr)
        ratio_ref[pl.ds(qbase, _QG), :] = jnp.concatenate(rows, axis=0)
        return carry

    lax.fori_loop(0, _N // _QG, gbody, 0)


def _ratio(cT, cC, x, gm):
    return pl.pallas_call(
        _ratio_kernel,
        grid=(2,),
        in_specs=[
            pl.BlockSpec((None, _N, 2), lambda b: (b, 0, 0)),
            pl.BlockSpec((None, 2, _N), lambda b: (b, 0, 0)),
            pl.BlockSpec((None, _N, _E), lambda b: (b, 0, 0)),
            pl.BlockSpec((None, 1, _E), lambda b: (b, 0, 0)),
        ],
        out_specs=pl.BlockSpec((None, _N, _E), lambda b: (b, 0, 0)),
        out_shape=jax.ShapeDtypeStruct((2, _N, _E), jnp.float32),
        scratch_shapes=[pltpu.VMEM((_QG * _NH, _E), jnp.float32)],
    )(cT, cC, x, gm)


def _select_kernel(ldR_ref, ldC_ref, x_ref, cT_ref, xo_ref, co_ref):
    ldrow = ldR_ref[0:1, :]
    iota = lax.broadcasted_iota(jnp.int32, (1, _N), 1)

    def body(i, carry):
        ldi = jnp.sum(ldC_ref[pl.ds(i, 1), :])
        beats = (ldrow > ldi) | ((ldrow == ldi) & (iota < i))
        r = jnp.sum(beats.astype(jnp.int32))

        @pl.when(r < _NKEEP)
        def _():
            xo_ref[pl.ds(r, 1), :] = x_ref[pl.ds(i, 1), :]
            co_ref[pl.ds(r, 1), :] = cT_ref[pl.ds(i, 1), :]

        return carry

    lax.fori_loop(0, _N, body, 0)


def _select(ld, x, cT):
    ldR = ld.reshape(2, 1, _N)
    ldC = ld.reshape(2, _N, 1)
    xo, co = pl.pallas_call(
        _select_kernel,
        grid=(2,),
        in_specs=[
            pl.BlockSpec((None, 1, _N), lambda b: (b, 0, 0)),
            pl.BlockSpec((None, _N, 1), lambda b: (b, 0, 0)),
            pl.BlockSpec((None, _N, _E), lambda b: (b, 0, 0)),
            pl.BlockSpec((None, _N, 2), lambda b: (b, 0, 0)),
        ],
        out_specs=[
            pl.BlockSpec((None, _NKEEP, _E), lambda b: (b, 0, 0)),
            pl.BlockSpec((None, _NKEEP, 2), lambda b: (b, 0, 0)),
        ],
        out_shape=[
            jax.ShapeDtypeStruct((2, _NKEEP, _E), jnp.float32),
            jax.ShapeDtypeStruct((2, _NKEEP, 2), jnp.float32),
        ],
    )(ldR, ldC, x, cT)
    return xo, jnp.transpose(co, (0, 2, 1))[:, :, :, None]


def kernel(x, coords):
    cT = jnp.transpose(coords[:, :, :, 0], (0, 2, 1))
    cC = coords[:, :, :, 0]
    gm = _gm(x)
    ratio = _ratio(cT, cC, x, gm)
    ld = jnp.sum(ratio, axis=-1)
    pad = coords[:, 0, :, 0] > 999.0
    ld = jnp.where(pad, ld - 10000.0, ld)
    x_out, coords_out = _select(ld, x, cT)
    return (x_out, coords_out, ld)


# speedup vs baseline: 7.8132x; 1.1164x over previous
import jax
import jax.numpy as jnp
from jax import lax
from jax.experimental import pallas as pl
from jax.experimental.pallas import tpu as pltpu

_NH = 32
_NKEEP = 1024
_N = 2048
_E = 256


def _gm_kernel(x_ref, gm_ref):
    gm_ref[...] = jnp.abs(jnp.mean(x_ref[...], axis=0, keepdims=True))


def _gm(x):
    return pl.pallas_call(
        _gm_kernel,
        grid=(2,),
        in_specs=[pl.BlockSpec((None, _N, _E), lambda b: (b, 0, 0))],
        out_specs=pl.BlockSpec((None, 1, _E), lambda b: (b, 0, 0)),
        out_shape=jax.ShapeDtypeStruct((2, 1, _E), jnp.float32),
    )(x)


_QG = 8


def _ratio_kernel(cT_ref, cC_ref, x_ref, gm_ref, ratio_ref, xnh_ref):
    ckx = cC_ref[0:1, :]
    cky = cC_ref[1:2, :]
    iota = lax.broadcasted_iota(jnp.int32, (_QG, _N), 1)
    gmr = gm_ref[...].reshape(1, _E)

    def gbody(g, carry):
        qbase = g * _QG
        cq = cT_ref[pl.ds(qbase, _QG), :]
        dx = cq[:, 0:1] - ckx
        dy = cq[:, 1:2] - cky
        d2 = dx * dx + dy * dy

        def kbody(k, d2c):
            m = jnp.min(d2c, axis=1, keepdims=True)
            cand = jnp.where(d2c == m, iota, _N)
            jcol = jnp.min(cand, axis=1, keepdims=True)
            js = [jnp.min(jcol[qq:qq + 1, 0:1]) for qq in range(_QG)]
            for qq in range(_QG):
                xnh_ref[pl.ds(qq * _NH + k, 1), :] = x_ref[pl.ds(js[qq], 1), :]
            return jnp.where(iota == jcol, jnp.inf, d2c)

        lax.fori_loop(0, _NH, kbody, d2)
        rows = []
        for qq in range(_QG):
            v = xnh_ref[qq * _NH:(qq + 1) * _NH, :]
            am = jnp.mean(v, axis=0, keepdims=True)
            cent = lax.square(lax.sub(v, am))
            ss = jnp.sum(cent, axis=0, keepdims=True)
            ls = jnp.sqrt(lax.div(ss, jnp.float32(_NH - 1)))
            rows.append(ls / gmr)
        ratio_ref[pl.ds(qbase, _QG), :] = jnp.concatenate(rows, axis=0)
        return carry

    lax.fori_loop(0, _N // _QG, gbody, 0)


def _ratio(cT, cC, x, gm):
    return pl.pallas_call(
        _ratio_kernel,
        grid=(2,),
        in_specs=[
            pl.BlockSpec((None, _N, 2), lambda b: (b, 0, 0)),
            pl.BlockSpec((None, 2, _N), lambda b: (b, 0, 0)),
            pl.BlockSpec((None, _N, _E), lambda b: (b, 0, 0)),
            pl.BlockSpec((None, 1, _E), lambda b: (b, 0, 0)),
        ],
        out_specs=pl.BlockSpec((None, _N, _E), lambda b: (b, 0, 0)),
        out_shape=jax.ShapeDtypeStruct((2, _N, _E), jnp.float32),
        scratch_shapes=[pltpu.VMEM((_QG * _NH, _E), jnp.float32)],
    )(cT, cC, x, gm)


def _select_kernel(ldR_ref, ldC_ref, x_ref, cT_ref, xo_ref, co_ref):
    ldrow = ldR_ref[0:1, :]
    iota = lax.broadcasted_iota(jnp.int32, (1, _N), 1)

    def body(g, carry):
        ibase = g * 8
        ldi8 = ldC_ref[pl.ds(ibase, 8), :]
        ii = lax.broadcasted_iota(jnp.int32, (8, 1), 0) + ibase
        beats = (ldrow > ldi8) | ((ldrow == ldi8) & (iota < ii))
        r8 = jnp.sum(beats.astype(jnp.int32), axis=1, keepdims=True)
        rs = [jnp.sum(r8[qq:qq + 1, 0:1]) for qq in range(8)]
        for qq in range(8):
            r = rs[qq]

            @pl.when(r < _NKEEP)
            def _():
                xo_ref[pl.ds(r, 1), :] = x_ref[pl.ds(ibase + qq, 1), :]
                co_ref[pl.ds(r, 1), :] = cT_ref[pl.ds(ibase + qq, 1), :]

        return carry

    lax.fori_loop(0, _N // 8, body, 0)


def _select(ld, x, cT):
    ldR = ld.reshape(2, 1, _N)
    ldC = ld.reshape(2, _N, 1)
    xo, co = pl.pallas_call(
        _select_kernel,
        grid=(2,),
        in_specs=[
            pl.BlockSpec((None, 1, _N), lambda b: (b, 0, 0)),
            pl.BlockSpec((None, _N, 1), lambda b: (b, 0, 0)),
            pl.BlockSpec((None, _N, _E), lambda b: (b, 0, 0)),
            pl.BlockSpec((None, _N, 2), lambda b: (b, 0, 0)),
        ],
        out_specs=[
            pl.BlockSpec((None, _NKEEP, _E), lambda b: (b, 0, 0)),
            pl.BlockSpec((None, _NKEEP, 2), lambda b: (b, 0, 0)),
        ],
        out_shape=[
            jax.ShapeDtypeStruct((2, _NKEEP, _E), jnp.float32),
            jax.ShapeDtypeStruct((2, _NKEEP, 2), jnp.float32),
        ],
    )(ldR, ldC, x, cT)
    return xo, jnp.transpose(co, (0, 2, 1))[:, :, :, None]


def kernel(x, coords):
    cT = jnp.transpose(coords[:, :, :, 0], (0, 2, 1))
    cC = coords[:, :, :, 0]
    gm = _gm(x)
    ratio = _ratio(cT, cC, x, gm)
    ld = jnp.sum(ratio, axis=-1)
    pad = coords[:, 0, :, 0] > 999.0
    ld = jnp.where(pad, ld - 10000.0, ld)
    x_out, coords_out = _select(ld, x, cT)
    return (x_out, coords_out, ld)


# speedup vs baseline: 8.7713x; 1.1226x over previous
import jax
import jax.numpy as jnp
from jax import lax
from jax.experimental import pallas as pl
from jax.experimental.pallas import tpu as pltpu

_NH = 32
_NKEEP = 1024
_N = 2048
_E = 256


def _gm_kernel(x_ref, gm_ref):
    gm_ref[...] = jnp.abs(jnp.mean(x_ref[...], axis=0, keepdims=True))


def _gm(x):
    return pl.pallas_call(
        _gm_kernel,
        grid=(2,),
        in_specs=[pl.BlockSpec((None, _N, _E), lambda b: (b, 0, 0))],
        out_specs=pl.BlockSpec((None, 1, _E), lambda b: (b, 0, 0)),
        out_shape=jax.ShapeDtypeStruct((2, 1, _E), jnp.float32),
    )(x)


_QG = 8


def _ratio_kernel(cT_ref, cC_ref, x_ref, gm_ref, ratio_ref, xnh_ref):
    ckx = cC_ref[0:1, :]
    cky = cC_ref[1:2, :]
    iota = lax.broadcasted_iota(jnp.int32, (_QG, _N), 1)
    gmr = gm_ref[...].reshape(1, _E)

    def gbody(g, carry):
        qbase = g * _QG
        cq = cT_ref[pl.ds(qbase, _QG), :]
        dx = cq[:, 0:1] - ckx
        dy = cq[:, 1:2] - cky
        d2 = dx * dx + dy * dy

        lane32 = lax.broadcasted_iota(jnp.int32, (_QG, _NH), 1)

        def kbody(k, carry):
            d2c, acc = carry
            m = jnp.min(d2c, axis=1, keepdims=True)
            cand = jnp.where(d2c == m, iota, _N)
            jcol = jnp.min(cand, axis=1, keepdims=True)
            acc = jnp.where(lane32 == k, jcol, acc)
            return jnp.where(iota == jcol, jnp.inf, d2c), acc

        _, acc = lax.fori_loop(
            0, _NH, kbody, (d2, jnp.zeros((_QG, _NH), jnp.int32)))
        for k in range(_NH):
            for qq in range(_QG):
                j = jnp.sum(acc[qq:qq + 1, k:k + 1])
                xnh_ref[qq * _NH + k:qq * _NH + k + 1, :] = x_ref[pl.ds(j, 1), :]
        rows = []
        for qq in range(_QG):
            v = xnh_ref[qq * _NH:(qq + 1) * _NH, :]
            am = jnp.mean(v, axis=0, keepdims=True)
            cent = lax.square(lax.sub(v, am))
            ss = jnp.sum(cent, axis=0, keepdims=True)
            ls = jnp.sqrt(lax.div(ss, jnp.float32(_NH - 1)))
            rows.append(ls / gmr)
        ratio_ref[pl.ds(qbase, _QG), :] = jnp.concatenate(rows, axis=0)
        return carry

    lax.fori_loop(0, _N // _QG, gbody, 0)


def _ratio(cT, cC, x, gm):
    return pl.pallas_call(
        _ratio_kernel,
        grid=(2,),
        in_specs=[
            pl.BlockSpec((None, _N, 2), lambda b: (b, 0, 0)),
            pl.BlockSpec((None, 2, _N), lambda b: (b, 0, 0)),
            pl.BlockSpec((None, _N, _E), lambda b: (b, 0, 0)),
            pl.BlockSpec((None, 1, _E), lambda b: (b, 0, 0)),
        ],
        out_specs=pl.BlockSpec((None, _N, _E), lambda b: (b, 0, 0)),
        out_shape=jax.ShapeDtypeStruct((2, _N, _E), jnp.float32),
        scratch_shapes=[pltpu.VMEM((_QG * _NH, _E), jnp.float32)],
    )(cT, cC, x, gm)


def _select_kernel(ldR_ref, ldC_ref, x_ref, cT_ref, xo_ref, co_ref):
    ldrow = ldR_ref[0:1, :]
    iota = lax.broadcasted_iota(jnp.int32, (1, _N), 1)

    def body(g, carry):
        ibase = g * 8
        ldi8 = ldC_ref[pl.ds(ibase, 8), :]
        ii = lax.broadcasted_iota(jnp.int32, (8, 1), 0) + ibase
        beats = (ldrow > ldi8) | ((ldrow == ldi8) & (iota < ii))
        r8 = jnp.sum(beats.astype(jnp.int32), axis=1, keepdims=True)
        rs = [jnp.sum(r8[qq:qq + 1, 0:1]) for qq in range(8)]
        for qq in range(8):
            r = rs[qq]

            @pl.when(r < _NKEEP)
            def _():
                xo_ref[pl.ds(r, 1), :] = x_ref[pl.ds(ibase + qq, 1), :]
                co_ref[pl.ds(r, 1), :] = cT_ref[pl.ds(ibase + qq, 1), :]

        return carry

    lax.fori_loop(0, _N // 8, body, 0)


def _select(ld, x, cT):
    ldR = ld.reshape(2, 1, _N)
    ldC = ld.reshape(2, _N, 1)
    xo, co = pl.pallas_call(
        _select_kernel,
        grid=(2,),
        in_specs=[
            pl.BlockSpec((None, 1, _N), lambda b: (b, 0, 0)),
            pl.BlockSpec((None, _N, 1), lambda b: (b, 0, 0)),
            pl.BlockSpec((None, _N, _E), lambda b: (b, 0, 0)),
            pl.BlockSpec((None, _N, 2), lambda b: (b, 0, 0)),
        ],
        out_specs=[
            pl.BlockSpec((None, _NKEEP, _E), lambda b: (b, 0, 0)),
            pl.BlockSpec((None, _NKEEP, 2), lambda b: (b, 0, 0)),
        ],
        out_shape=[
            jax.ShapeDtypeStruct((2, _NKEEP, _E), jnp.float32),
            jax.ShapeDtypeStruct((2, _NKEEP, 2), jnp.float32),
        ],
    )(ldR, ldC, x, cT)
    return xo, jnp.transpose(co, (0, 2, 1))[:, :, :, None]


def kernel(x, coords):
    cT = jnp.transpose(coords[:, :, :, 0], (0, 2, 1))
    cC = coords[:, :, :, 0]
    gm = _gm(x)
    ratio = _ratio(cT, cC, x, gm)
    ld = jnp.sum(ratio, axis=-1)
    pad = coords[:, 0, :, 0] > 999.0
    ld = jnp.where(pad, ld - 10000.0, ld)
    x_out, coords_out = _select(ld, x, cT)
    return (x_out, coords_out, ld)


# speedup vs baseline: 9.2695x; 1.0568x over previous
import jax
import jax.numpy as jnp
from jax import lax
from jax.experimental import pallas as pl
from jax.experimental.pallas import tpu as pltpu

_NH = 32
_NKEEP = 1024
_N = 2048
_E = 256


def _gm_kernel(x_ref, gm_ref):
    gm_ref[...] = jnp.abs(jnp.mean(x_ref[...], axis=0, keepdims=True))


def _gm(x):
    return pl.pallas_call(
        _gm_kernel,
        grid=(2,),
        in_specs=[pl.BlockSpec((None, _N, _E), lambda b: (b, 0, 0))],
        out_specs=pl.BlockSpec((None, 1, _E), lambda b: (b, 0, 0)),
        out_shape=jax.ShapeDtypeStruct((2, 1, _E), jnp.float32),
    )(x)


_QG = 8


def _ratio_kernel(cT_ref, cC_ref, x_ref, gm_ref, ratio_ref, xnh_ref):
    ckx = cC_ref[0:1, :]
    cky = cC_ref[1:2, :]
    iota = lax.broadcasted_iota(jnp.int32, (_QG, _N), 1)
    gmr = gm_ref[...].reshape(1, _E)

    def gbody(g, carry):
        qbase = g * _QG
        cq = cT_ref[pl.ds(qbase, _QG), :]
        dx = cq[:, 0:1] - ckx
        dy = cq[:, 1:2] - cky
        d2 = dx * dx + dy * dy

        d2c = d2
        jcols = []
        for k in range(_NH):
            m = jnp.min(d2c, axis=1, keepdims=True)
            cand = jnp.where(d2c == m, iota, _N)
            jcol = jnp.min(cand, axis=1, keepdims=True)
            jcols.append(jcol)
            d2c = jnp.where(iota == jcol, jnp.inf, d2c)
        for k in range(_NH):
            for qq in range(_QG):
                j = jnp.sum(jcols[k][qq:qq + 1, 0:1])
                xnh_ref[qq * _NH + k:qq * _NH + k + 1, :] = x_ref[pl.ds(j, 1), :]
        rows = []
        for qq in range(_QG):
            v = xnh_ref[qq * _NH:(qq + 1) * _NH, :]
            am = jnp.mean(v, axis=0, keepdims=True)
            cent = lax.square(lax.sub(v, am))
            ss = jnp.sum(cent, axis=0, keepdims=True)
            ls = jnp.sqrt(lax.div(ss, jnp.float32(_NH - 1)))
            rows.append(ls / gmr)
        ratio_ref[pl.ds(qbase, _QG), :] = jnp.concatenate(rows, axis=0)
        return carry

    lax.fori_loop(0, _N // _QG, gbody, 0)


def _ratio(cT, cC, x, gm):
    return pl.pallas_call(
        _ratio_kernel,
        grid=(2,),
        in_specs=[
            pl.BlockSpec((None, _N, 2), lambda b: (b, 0, 0)),
            pl.BlockSpec((None, 2, _N), lambda b: (b, 0, 0)),
            pl.BlockSpec((None, _N, _E), lambda b: (b, 0, 0)),
            pl.BlockSpec((None, 1, _E), lambda b: (b, 0, 0)),
        ],
        out_specs=pl.BlockSpec((None, _N, _E), lambda b: (b, 0, 0)),
        out_shape=jax.ShapeDtypeStruct((2, _N, _E), jnp.float32),
        scratch_shapes=[pltpu.VMEM((_QG * _NH, _E), jnp.float32)],
    )(cT, cC, x, gm)


def _select_kernel(ldR_ref, ldC_ref, x_ref, cT_ref, xo_ref, co_ref):
    ldrow = ldR_ref[0:1, :]
    iota = lax.broadcasted_iota(jnp.int32, (1, _N), 1)

    def body(g, carry):
        ibase = g * 8
        ldi8 = ldC_ref[pl.ds(ibase, 8), :]
        ii = lax.broadcasted_iota(jnp.int32, (8, 1), 0) + ibase
        beats = (ldrow > ldi8) | ((ldrow == ldi8) & (iota < ii))
        r8 = jnp.sum(beats.astype(jnp.int32), axis=1, keepdims=True)
        rs = [jnp.sum(r8[qq:qq + 1, 0:1]) for qq in range(8)]
        for qq in range(8):
            r = rs[qq]

            @pl.when(r < _NKEEP)
            def _():
                xo_ref[pl.ds(r, 1), :] = x_ref[pl.ds(ibase + qq, 1), :]
                co_ref[pl.ds(r, 1), :] = cT_ref[pl.ds(ibase + qq, 1), :]

        return carry

    lax.fori_loop(0, _N // 8, body, 0)


def _select(ld, x, cT):
    ldR = ld.reshape(2, 1, _N)
    ldC = ld.reshape(2, _N, 1)
    xo, co = pl.pallas_call(
        _select_kernel,
        grid=(2,),
        in_specs=[
            pl.BlockSpec((None, 1, _N), lambda b: (b, 0, 0)),
            pl.BlockSpec((None, _N, 1), lambda b: (b, 0, 0)),
            pl.BlockSpec((None, _N, _E), lambda b: (b, 0, 0)),
            pl.BlockSpec((None, _N, 2), lambda b: (b, 0, 0)),
        ],
        out_specs=[
            pl.BlockSpec((None, _NKEEP, _E), lambda b: (b, 0, 0)),
            pl.BlockSpec((None, _NKEEP, 2), lambda b: (b, 0, 0)),
        ],
        out_shape=[
            jax.ShapeDtypeStruct((2, _NKEEP, _E), jnp.float32),
            jax.ShapeDtypeStruct((2, _NKEEP, 2), jnp.float32),
        ],
    )(ldR, ldC, x, cT)
    return xo, jnp.transpose(co, (0, 2, 1))[:, :, :, None]


def kernel(x, coords):
    cT = jnp.transpose(coords[:, :, :, 0], (0, 2, 1))
    cC = coords[:, :, :, 0]
    gm = _gm(x)
    ratio = _ratio(cT, cC, x, gm)
    ld = jnp.sum(ratio, axis=-1)
    pad = coords[:, 0, :, 0] > 999.0
    ld = jnp.where(pad, ld - 10000.0, ld)
    x_out, coords_out = _select(ld, x, cT)
    return (x_out, coords_out, ld)


# speedup vs baseline: 17.7059x; 1.9101x over previous
import jax
import jax.numpy as jnp
from jax import lax
from jax.experimental import pallas as pl
from jax.experimental.pallas import tpu as pltpu

_NH = 32
_NKEEP = 1024
_N = 2048
_E = 256


def _gm_kernel(x_ref, gm_ref):
    gm_ref[...] = jnp.abs(jnp.mean(x_ref[...], axis=0, keepdims=True))


def _gm(x):
    return pl.pallas_call(
        _gm_kernel,
        grid=(2,),
        in_specs=[pl.BlockSpec((None, _N, _E), lambda b: (b, 0, 0))],
        out_specs=pl.BlockSpec((None, 1, _E), lambda b: (b, 0, 0)),
        out_shape=jax.ShapeDtypeStruct((2, 1, _E), jnp.float32),
    )(x)


_QG = 8


def _ratio_kernel(cT_ref, cC_ref, x_ref, gm_ref, ratio_ref, xnh_ref):
    ckx = cC_ref[0:1, :]
    cky = cC_ref[1:2, :]
    iota = lax.broadcasted_iota(jnp.int32, (_QG, _N), 1)
    gmr = gm_ref[...].reshape(1, _E)

    def gbody(g, carry):
        qbase = g * (2 * _QG)
        cqA = cT_ref[pl.ds(qbase, _QG), :]
        cqB = cT_ref[pl.ds(qbase + _QG, _QG), :]
        dxA = cqA[:, 0:1] - ckx
        dyA = cqA[:, 1:2] - cky
        d2A = dxA * dxA + dyA * dyA
        dxB = cqB[:, 0:1] - ckx
        dyB = cqB[:, 1:2] - cky
        d2B = dxB * dxB + dyB * dyB

        for k in range(_NH):
            mA = jnp.min(d2A, axis=1, keepdims=True)
            mB = jnp.min(d2B, axis=1, keepdims=True)
            candA = jnp.where(d2A == mA, iota, _N)
            candB = jnp.where(d2B == mB, iota, _N)
            jA = jnp.min(candA, axis=1, keepdims=True)
            jB = jnp.min(candB, axis=1, keepdims=True)
            for qq in range(_QG):
                j1 = jnp.sum(jA[qq:qq + 1, 0:1])
                xnh_ref[qq * _NH + k:qq * _NH + k + 1, :] = x_ref[pl.ds(j1, 1), :]
                j2 = jnp.sum(jB[qq:qq + 1, 0:1])
                xnh_ref[(_QG + qq) * _NH + k:(_QG + qq) * _NH + k + 1, :] = (
                    x_ref[pl.ds(j2, 1), :])
            d2A = jnp.where(iota == jA, jnp.inf, d2A)
            d2B = jnp.where(iota == jB, jnp.inf, d2B)
        rows = []
        for qq in range(2 * _QG):
            v = xnh_ref[qq * _NH:(qq + 1) * _NH, :]
            am = jnp.mean(v, axis=0, keepdims=True)
            cent = lax.square(lax.sub(v, am))
            ss = jnp.sum(cent, axis=0, keepdims=True)
            ls = jnp.sqrt(lax.div(ss, jnp.float32(_NH - 1)))
            rows.append(ls / gmr)
        ratio_ref[pl.ds(qbase, 2 * _QG), :] = jnp.concatenate(rows, axis=0)
        return carry

    lax.fori_loop(0, _N // (2 * _QG), gbody, 0)


def _ratio(cT, cC, x, gm):
    return pl.pallas_call(
        _ratio_kernel,
        grid=(2,),
        in_specs=[
            pl.BlockSpec((None, _N, 2), lambda b: (b, 0, 0)),
            pl.BlockSpec((None, 2, _N), lambda b: (b, 0, 0)),
            pl.BlockSpec((None, _N, _E), lambda b: (b, 0, 0)),
            pl.BlockSpec((None, 1, _E), lambda b: (b, 0, 0)),
        ],
        out_specs=pl.BlockSpec((None, _N, _E), lambda b: (b, 0, 0)),
        out_shape=jax.ShapeDtypeStruct((2, _N, _E), jnp.float32),
        scratch_shapes=[pltpu.VMEM((2 * _QG * _NH, _E), jnp.float32)],
    )(cT, cC, x, gm)


def _select_kernel(ldR_ref, ldC_ref, x_ref, cT_ref, xo_ref, co_ref):
    ldrow = ldR_ref[0:1, :]
    iota = lax.broadcasted_iota(jnp.int32, (1, _N), 1)

    def body(g, carry):
        ibase = g * 8
        ldi8 = ldC_ref[pl.ds(ibase, 8), :]
        ii = lax.broadcasted_iota(jnp.int32, (8, 1), 0) + ibase
        beats = (ldrow > ldi8) | ((ldrow == ldi8) & (iota < ii))
        r8 = jnp.sum(beats.astype(jnp.int32), axis=1, keepdims=True)
        rs = [jnp.sum(r8[qq:qq + 1, 0:1]) for qq in range(8)]
        for qq in range(8):
            r = rs[qq]

            @pl.when(r < _NKEEP)
            def _():
                xo_ref[pl.ds(r, 1), :] = x_ref[pl.ds(ibase + qq, 1), :]
                co_ref[pl.ds(r, 1), :] = cT_ref[pl.ds(ibase + qq, 1), :]

        return carry

    lax.fori_loop(0, _N // 8, body, 0)


def _select(ld, x, cT):
    ldR = ld.reshape(2, 1, _N)
    ldC = ld.reshape(2, _N, 1)
    xo, co = pl.pallas_call(
        _select_kernel,
        grid=(2,),
        in_specs=[
            pl.BlockSpec((None, 1, _N), lambda b: (b, 0, 0)),
            pl.BlockSpec((None, _N, 1), lambda b: (b, 0, 0)),
            pl.BlockSpec((None, _N, _E), lambda b: (b, 0, 0)),
            pl.BlockSpec((None, _N, 2), lambda b: (b, 0, 0)),
        ],
        out_specs=[
            pl.BlockSpec((None, _NKEEP, _E), lambda b: (b, 0, 0)),
            pl.BlockSpec((None, _NKEEP, 2), lambda b: (b, 0, 0)),
        ],
        out_shape=[
            jax.ShapeDtypeStruct((2, _NKEEP, _E), jnp.float32),
            jax.ShapeDtypeStruct((2, _NKEEP, 2), jnp.float32),
        ],
    )(ldR, ldC, x, cT)
    return xo, jnp.transpose(co, (0, 2, 1))[:, :, :, None]


def kernel(x, coords):
    cT = jnp.transpose(coords[:, :, :, 0], (0, 2, 1))
    cC = coords[:, :, :, 0]
    gm = _gm(x)
    ratio = _ratio(cT, cC, x, gm)
    ld = jnp.sum(ratio, axis=-1)
    pad = coords[:, 0, :, 0] > 999.0
    ld = jnp.where(pad, ld - 10000.0, ld)
    x_out, coords_out = _select(ld, x, cT)
    return (x_out, coords_out, ld)


# speedup vs baseline: 31.7961x; 1.7958x over previous
import jax
import jax.numpy as jnp
from jax import lax
from jax.experimental import pallas as pl
from jax.experimental.pallas import tpu as pltpu

_NH = 32
_NKEEP = 1024
_N = 2048
_E = 256


def _gm_kernel(x_ref, gm_ref):
    gm_ref[...] = jnp.abs(jnp.mean(x_ref[...], axis=0, keepdims=True))


def _gm(x):
    return pl.pallas_call(
        _gm_kernel,
        grid=(2,),
        in_specs=[pl.BlockSpec((None, _N, _E), lambda b: (b, 0, 0))],
        out_specs=pl.BlockSpec((None, 1, _E), lambda b: (b, 0, 0)),
        out_shape=jax.ShapeDtypeStruct((2, 1, _E), jnp.float32),
    )(x)


_QG = 8
_NCH = 4


def _ratio_kernel(cT_ref, cC_ref, x_ref, gm_ref, ratio_ref, xnh_ref):
    ckx = cC_ref[0:1, :]
    cky = cC_ref[1:2, :]
    iota = lax.broadcasted_iota(jnp.int32, (_QG, _N), 1)
    gmr = gm_ref[...].reshape(1, _E)

    def gbody(g, carry):
        qbase = g * (_NCH * _QG)
        d2s = []
        for c in range(_NCH):
            cq = cT_ref[pl.ds(qbase + c * _QG, _QG), :]
            dx = cq[:, 0:1] - ckx
            dy = cq[:, 1:2] - cky
            d2s.append(dx * dx + dy * dy)

        for k in range(_NH):
            ms = [jnp.min(d2s[c], axis=1, keepdims=True) for c in range(_NCH)]
            cands = [jnp.where(d2s[c] == ms[c], iota, _N) for c in range(_NCH)]
            js = [jnp.min(cands[c], axis=1, keepdims=True) for c in range(_NCH)]
            for c in range(_NCH):
                for qq in range(_QG):
                    j1 = jnp.sum(js[c][qq:qq + 1, 0:1])
                    row = (c * _QG + qq) * _NH + k
                    xnh_ref[row:row + 1, :] = x_ref[pl.ds(j1, 1), :]
            for c in range(_NCH):
                d2s[c] = jnp.where(iota == js[c], jnp.inf, d2s[c])
        rows = []
        for qq in range(_NCH * _QG):
            v = xnh_ref[qq * _NH:(qq + 1) * _NH, :]
            am = jnp.mean(v, axis=0, keepdims=True)
            cent = lax.square(lax.sub(v, am))
            ss = jnp.sum(cent, axis=0, keepdims=True)
            ls = jnp.sqrt(lax.div(ss, jnp.float32(_NH - 1)))
            rows.append(ls / gmr)
        ratio_ref[pl.ds(qbase, _NCH * _QG), :] = jnp.concatenate(rows, axis=0)
        return carry

    lax.fori_loop(0, _N // (_NCH * _QG), gbody, 0)


def _ratio(cT, cC, x, gm):
    return pl.pallas_call(
        _ratio_kernel,
        grid=(2,),
        in_specs=[
            pl.BlockSpec((None, _N, 2), lambda b: (b, 0, 0)),
            pl.BlockSpec((None, 2, _N), lambda b: (b, 0, 0)),
            pl.BlockSpec((None, _N, _E), lambda b: (b, 0, 0)),
            pl.BlockSpec((None, 1, _E), lambda b: (b, 0, 0)),
        ],
        out_specs=pl.BlockSpec((None, _N, _E), lambda b: (b, 0, 0)),
        out_shape=jax.ShapeDtypeStruct((2, _N, _E), jnp.float32),
        scratch_shapes=[pltpu.VMEM((_NCH * _QG * _NH, _E), jnp.float32)],
    )(cT, cC, x, gm)


def _select_kernel(ldR_ref, ldC_ref, x_ref, cT_ref, xo_ref, co_ref):
    ldrow = ldR_ref[0:1, :]
    iota = lax.broadcasted_iota(jnp.int32, (1, _N), 1)

    def body(g, carry):
        ibase = g * 8
        ldi8 = ldC_ref[pl.ds(ibase, 8), :]
        ii = lax.broadcasted_iota(jnp.int32, (8, 1), 0) + ibase
        beats = (ldrow > ldi8) | ((ldrow == ldi8) & (iota < ii))
        r8 = jnp.sum(beats.astype(jnp.int32), axis=1, keepdims=True)
        rs = [jnp.sum(r8[qq:qq + 1, 0:1]) for qq in range(8)]
        for qq in range(8):
            r = rs[qq]

            @pl.when(r < _NKEEP)
            def _():
                xo_ref[pl.ds(r, 1), :] = x_ref[pl.ds(ibase + qq, 1), :]
                co_ref[pl.ds(r, 1), :] = cT_ref[pl.ds(ibase + qq, 1), :]

        return carry

    lax.fori_loop(0, _N // 8, body, 0)


def _select(ld, x, cT):
    ldR = ld.reshape(2, 1, _N)
    ldC = ld.reshape(2, _N, 1)
    xo, co = pl.pallas_call(
        _select_kernel,
        grid=(2,),
        in_specs=[
            pl.BlockSpec((None, 1, _N), lambda b: (b, 0, 0)),
            pl.BlockSpec((None, _N, 1), lambda b: (b, 0, 0)),
            pl.BlockSpec((None, _N, _E), lambda b: (b, 0, 0)),
            pl.BlockSpec((None, _N, 2), lambda b: (b, 0, 0)),
        ],
        out_specs=[
            pl.BlockSpec((None, _NKEEP, _E), lambda b: (b, 0, 0)),
            pl.BlockSpec((None, _NKEEP, 2), lambda b: (b, 0, 0)),
        ],
        out_shape=[
            jax.ShapeDtypeStruct((2, _NKEEP, _E), jnp.float32),
            jax.ShapeDtypeStruct((2, _NKEEP, 2), jnp.float32),
        ],
    )(ldR, ldC, x, cT)
    return xo, jnp.transpose(co, (0, 2, 1))[:, :, :, None]


def kernel(x, coords):
    cT = jnp.transpose(coords[:, :, :, 0], (0, 2, 1))
    cC = coords[:, :, :, 0]
    gm = _gm(x)
    ratio = _ratio(cT, cC, x, gm)
    ld = jnp.sum(ratio, axis=-1)
    pad = coords[:, 0, :, 0] > 999.0
    ld = jnp.where(pad, ld - 10000.0, ld)
    x_out, coords_out = _select(ld, x, cT)
    return (x_out, coords_out, ld)


# speedup vs baseline: 54.1544x; 1.7032x over previous
import jax
import jax.numpy as jnp
from jax import lax
from jax.experimental import pallas as pl
from jax.experimental.pallas import tpu as pltpu

_NH = 32
_NKEEP = 1024
_N = 2048
_E = 256


def _gm_kernel(x_ref, gm_ref):
    gm_ref[...] = jnp.abs(jnp.mean(x_ref[...], axis=0, keepdims=True))


def _gm(x):
    return pl.pallas_call(
        _gm_kernel,
        grid=(2,),
        in_specs=[pl.BlockSpec((None, _N, _E), lambda b: (b, 0, 0))],
        out_specs=pl.BlockSpec((None, 1, _E), lambda b: (b, 0, 0)),
        out_shape=jax.ShapeDtypeStruct((2, 1, _E), jnp.float32),
    )(x)


_QG = 8
_NCH = 8


def _ratio_kernel(cT_ref, cC_ref, x_ref, gm_ref, ratio_ref, xnh_ref):
    ckx = cC_ref[0:1, :]
    cky = cC_ref[1:2, :]
    iota = lax.broadcasted_iota(jnp.int32, (_QG, _N), 1)
    gmr = gm_ref[...].reshape(1, _E)

    def gbody(g, carry):
        qbase = g * (_NCH * _QG)
        d2s = []
        for c in range(_NCH):
            cq = cT_ref[pl.ds(qbase + c * _QG, _QG), :]
            dx = cq[:, 0:1] - ckx
            dy = cq[:, 1:2] - cky
            d2s.append(dx * dx + dy * dy)

        for k in range(_NH):
            ms = [jnp.min(d2s[c], axis=1, keepdims=True) for c in range(_NCH)]
            cands = [jnp.where(d2s[c] == ms[c], iota, _N) for c in range(_NCH)]
            js = [jnp.min(cands[c], axis=1, keepdims=True) for c in range(_NCH)]
            for c in range(_NCH):
                for qq in range(_QG):
                    j1 = jnp.sum(js[c][qq:qq + 1, 0:1])
                    row = (c * _QG + qq) * _NH + k
                    xnh_ref[row:row + 1, :] = x_ref[pl.ds(j1, 1), :]
            for c in range(_NCH):
                d2s[c] = jnp.where(iota == js[c], jnp.inf, d2s[c])
        rows = []
        for qq in range(_NCH * _QG):
            v = xnh_ref[qq * _NH:(qq + 1) * _NH, :]
            am = jnp.mean(v, axis=0, keepdims=True)
            cent = lax.square(lax.sub(v, am))
            ss = jnp.sum(cent, axis=0, keepdims=True)
            ls = jnp.sqrt(lax.div(ss, jnp.float32(_NH - 1)))
            rows.append(ls / gmr)
        ratio_ref[pl.ds(qbase, _NCH * _QG), :] = jnp.concatenate(rows, axis=0)
        return carry

    lax.fori_loop(0, _N // (_NCH * _QG), gbody, 0)


def _ratio(cT, cC, x, gm):
    return pl.pallas_call(
        _ratio_kernel,
        grid=(2,),
        in_specs=[
            pl.BlockSpec((None, _N, 2), lambda b: (b, 0, 0)),
            pl.BlockSpec((None, 2, _N), lambda b: (b, 0, 0)),
            pl.BlockSpec((None, _N, _E), lambda b: (b, 0, 0)),
            pl.BlockSpec((None, 1, _E), lambda b: (b, 0, 0)),
        ],
        out_specs=pl.BlockSpec((None, _N, _E), lambda b: (b, 0, 0)),
        out_shape=jax.ShapeDtypeStruct((2, _N, _E), jnp.float32),
        scratch_shapes=[pltpu.VMEM((_NCH * _QG * _NH, _E), jnp.float32)],
    )(cT, cC, x, gm)


def _select_kernel(ldR_ref, ldC_ref, x_ref, cT_ref, xo_ref, co_ref):
    ldrow = ldR_ref[0:1, :]
    iota = lax.broadcasted_iota(jnp.int32, (1, _N), 1)

    def body(g, carry):
        ibase = g * 8
        ldi8 = ldC_ref[pl.ds(ibase, 8), :]
        ii = lax.broadcasted_iota(jnp.int32, (8, 1), 0) + ibase
        beats = (ldrow > ldi8) | ((ldrow == ldi8) & (iota < ii))
        r8 = jnp.sum(beats.astype(jnp.int32), axis=1, keepdims=True)
        rs = [jnp.sum(r8[qq:qq + 1, 0:1]) for qq in range(8)]
        for qq in range(8):
            r = rs[qq]

            @pl.when(r < _NKEEP)
            def _():
                xo_ref[pl.ds(r, 1), :] = x_ref[pl.ds(ibase + qq, 1), :]
                co_ref[pl.ds(r, 1), :] = cT_ref[pl.ds(ibase + qq, 1), :]

        return carry

    lax.fori_loop(0, _N // 8, body, 0)


def _select(ld, x, cT):
    ldR = ld.reshape(2, 1, _N)
    ldC = ld.reshape(2, _N, 1)
    xo, co = pl.pallas_call(
        _select_kernel,
        grid=(2,),
        in_specs=[
            pl.BlockSpec((None, 1, _N), lambda b: (b, 0, 0)),
            pl.BlockSpec((None, _N, 1), lambda b: (b, 0, 0)),
            pl.BlockSpec((None, _N, _E), lambda b: (b, 0, 0)),
            pl.BlockSpec((None, _N, 2), lambda b: (b, 0, 0)),
        ],
        out_specs=[
            pl.BlockSpec((None, _NKEEP, _E), lambda b: (b, 0, 0)),
            pl.BlockSpec((None, _NKEEP, 2), lambda b: (b, 0, 0)),
        ],
        out_shape=[
            jax.ShapeDtypeStruct((2, _NKEEP, _E), jnp.float32),
            jax.ShapeDtypeStruct((2, _NKEEP, 2), jnp.float32),
        ],
    )(ldR, ldC, x, cT)
    return xo, jnp.transpose(co, (0, 2, 1))[:, :, :, None]


def kernel(x, coords):
    cT = jnp.transpose(coords[:, :, :, 0], (0, 2, 1))
    cC = coords[:, :, :, 0]
    gm = _gm(x)
    ratio = _ratio(cT, cC, x, gm)
    ld = jnp.sum(ratio, axis=-1)
    pad = coords[:, 0, :, 0] > 999.0
    ld = jnp.where(pad, ld - 10000.0, ld)
    x_out, coords_out = _select(ld, x, cT)
    return (x_out, coords_out, ld)


# speedup vs baseline: 64.9046x; 1.1985x over previous
import jax
import jax.numpy as jnp
from jax import lax
from jax.experimental import pallas as pl
from jax.experimental.pallas import tpu as pltpu

_NH = 32
_NKEEP = 1024
_N = 2048
_E = 256


def _gm_kernel(x_ref, gm_ref):
    gm_ref[...] = jnp.abs(jnp.mean(x_ref[...], axis=0, keepdims=True))


def _gm(x):
    return pl.pallas_call(
        _gm_kernel,
        grid=(2,),
        in_specs=[pl.BlockSpec((None, _N, _E), lambda b: (b, 0, 0))],
        out_specs=pl.BlockSpec((None, 1, _E), lambda b: (b, 0, 0)),
        out_shape=jax.ShapeDtypeStruct((2, 1, _E), jnp.float32),
    )(x)


_QG = 8
_NCH = 16


def _ratio_kernel(cT_ref, cC_ref, x_ref, gm_ref, ratio_ref, xnh_ref):
    ckx = cC_ref[0:1, :]
    cky = cC_ref[1:2, :]
    iota = lax.broadcasted_iota(jnp.int32, (_QG, _N), 1)
    gmr = gm_ref[...].reshape(1, _E)

    def gbody(g, carry):
        qbase = g * (_NCH * _QG)
        d2s = []
        for c in range(_NCH):
            cq = cT_ref[pl.ds(qbase + c * _QG, _QG), :]
            dx = cq[:, 0:1] - ckx
            dy = cq[:, 1:2] - cky
            d2s.append(dx * dx + dy * dy)

        for k in range(_NH):
            ms = [jnp.min(d2s[c], axis=1, keepdims=True) for c in range(_NCH)]
            cands = [jnp.where(d2s[c] == ms[c], iota, _N) for c in range(_NCH)]
            js = [jnp.min(cands[c], axis=1, keepdims=True) for c in range(_NCH)]
            for c in range(_NCH):
                for qq in range(_QG):
                    j1 = jnp.sum(js[c][qq:qq + 1, 0:1])
                    row = (c * _QG + qq) * _NH + k
                    xnh_ref[row:row + 1, :] = x_ref[pl.ds(j1, 1), :]
            for c in range(_NCH):
                d2s[c] = jnp.where(iota == js[c], jnp.inf, d2s[c])
        rows = []
        for qq in range(_NCH * _QG):
            v = xnh_ref[qq * _NH:(qq + 1) * _NH, :]
            am = jnp.mean(v, axis=0, keepdims=True)
            cent = lax.square(lax.sub(v, am))
            ss = jnp.sum(cent, axis=0, keepdims=True)
            ls = jnp.sqrt(lax.div(ss, jnp.float32(_NH - 1)))
            rows.append(ls / gmr)
        ratio_ref[pl.ds(qbase, _NCH * _QG), :] = jnp.concatenate(rows, axis=0)
        return carry

    lax.fori_loop(0, _N // (_NCH * _QG), gbody, 0)


def _ratio(cT, cC, x, gm):
    return pl.pallas_call(
        _ratio_kernel,
        grid=(2,),
        in_specs=[
            pl.BlockSpec((None, _N, 2), lambda b: (b, 0, 0)),
            pl.BlockSpec((None, 2, _N), lambda b: (b, 0, 0)),
            pl.BlockSpec((None, _N, _E), lambda b: (b, 0, 0)),
            pl.BlockSpec((None, 1, _E), lambda b: (b, 0, 0)),
        ],
        out_specs=pl.BlockSpec((None, _N, _E), lambda b: (b, 0, 0)),
        out_shape=jax.ShapeDtypeStruct((2, _N, _E), jnp.float32),
        scratch_shapes=[pltpu.VMEM((_NCH * _QG * _NH, _E), jnp.float32)],
    )(cT, cC, x, gm)


def _select_kernel(ldR_ref, ldC_ref, x_ref, cT_ref, xo_ref, co_ref):
    ldrow = ldR_ref[0:1, :]
    iota = lax.broadcasted_iota(jnp.int32, (1, _N), 1)

    def body(g, carry):
        ibase = g * 8
        ldi8 = ldC_ref[pl.ds(ibase, 8), :]
        ii = lax.broadcasted_iota(jnp.int32, (8, 1), 0) + ibase
        beats = (ldrow > ldi8) | ((ldrow == ldi8) & (iota < ii))
        r8 = jnp.sum(beats.astype(jnp.int32), axis=1, keepdims=True)
        rs = [jnp.sum(r8[qq:qq + 1, 0:1]) for qq in range(8)]
        for qq in range(8):
            r = rs[qq]

            @pl.when(r < _NKEEP)
            def _():
                xo_ref[pl.ds(r, 1), :] = x_ref[pl.ds(ibase + qq, 1), :]
                co_ref[pl.ds(r, 1), :] = cT_ref[pl.ds(ibase + qq, 1), :]

        return carry

    lax.fori_loop(0, _N // 8, body, 0)


def _select(ld, x, cT):
    ldR = ld.reshape(2, 1, _N)
    ldC = ld.reshape(2, _N, 1)
    xo, co = pl.pallas_call(
        _select_kernel,
        grid=(2,),
        in_specs=[
            pl.BlockSpec((None, 1, _N), lambda b: (b, 0, 0)),
            pl.BlockSpec((None, _N, 1), lambda b: (b, 0, 0)),
            pl.BlockSpec((None, _N, _E), lambda b: (b, 0, 0)),
            pl.BlockSpec((None, _N, 2), lambda b: (b, 0, 0)),
        ],
        out_specs=[
            pl.BlockSpec((None, _NKEEP, _E), lambda b: (b, 0, 0)),
            pl.BlockSpec((None, _NKEEP, 2), lambda b: (b, 0, 0)),
        ],
        out_shape=[
            jax.ShapeDtypeStruct((2, _NKEEP, _E), jnp.float32),
            jax.ShapeDtypeStruct((2, _NKEEP, 2), jnp.float32),
        ],
    )(ldR, ldC, x, cT)
    return xo, jnp.transpose(co, (0, 2, 1))[:, :, :, None]


def kernel(x, coords):
    cT = jnp.transpose(coords[:, :, :, 0], (0, 2, 1))
    cC = coords[:, :, :, 0]
    gm = _gm(x)
    ratio = _ratio(cT, cC, x, gm)
    ld = jnp.sum(ratio, axis=-1)
    pad = coords[:, 0, :, 0] > 999.0
    ld = jnp.where(pad, ld - 10000.0, ld)
    x_out, coords_out = _select(ld, x, cT)
    return (x_out, coords_out, ld)
